# baseline (speedup 1.0000x reference)
import functools

import jax
import jax.numpy as jnp
from jax import lax
from jax.experimental import pallas as pl
from jax.experimental.pallas import tpu as pltpu
from jax.experimental.pallas import tpu_sc as plsc


def kernel(indices, table):
    B, S = indices.shape
    V, D = table.shape
    N = B * S

    info = plsc.get_sparse_core_info()
    NC, NS = info.num_cores, info.num_subcores
    NW = NC * NS
    CHUNK = 128
    NBUF = 4
    assert N % (NW * CHUNK) == 0
    n_chunks = N // (NW * CHUNK)
    assert n_chunks % NBUF == 0 and n_chunks >= 3 * NBUF

    idx3 = indices.reshape(NW, n_chunks, CHUNK)

    mesh = plsc.VectorSubcoreMesh(core_axis_name="c", subcore_axis_name="s")

    @functools.partial(
        pl.kernel,
        mesh=mesh,
        out_type=jax.ShapeDtypeStruct((N, D), jnp.float32),
        scratch_types=(
            [pltpu.VMEM((n_chunks, CHUNK), jnp.int32)]
            + [pltpu.VMEM((CHUNK, D), jnp.float32)] * NBUF
            + [pltpu.SemaphoreType.DMA] * (2 * NBUF)
        ),
    )
    def sc_gather(idx_hbm, table_hbm, out_hbm, idx_v, *bufs_and_sems):
        bufs = bufs_and_sems[:NBUF]
        gs = bufs_and_sems[NBUF:2 * NBUF]
        ws = bufs_and_sems[2 * NBUF:]
        wid = lax.axis_index("s") * NC + lax.axis_index("c")
        base = wid * (n_chunks * CHUNK)
        pltpu.sync_copy(idx_hbm.at[wid], idx_v)

        def gather(j, b):
            return pltpu.make_async_copy(
                table_hbm.at[idx_v.at[j]], bufs[b], gs[b])

        def write(j, b):
            return pltpu.make_async_copy(
                bufs[b], out_hbm.at[pl.ds(base + j * CHUNK, CHUNK)], ws[b])

        def step(j, b):
            write(j - 2, (b + 2) % NBUF).wait()
            gather(j + 2, (b + 2) % NBUF).start()
            gather(j, b).wait()
            write(j, b).start()

        gather(0, 0).start()
        gather(1, 1).start()
        gather(0, 0).wait()
        write(0, 0).start()
        gather(2, 2).start()
        gather(1, 1).wait()
        write(1, 1).start()
        gather(3, 3).start()

        step(2, 2)
        step(3, 3)

        def body(g, carry):
            j0 = NBUF * g
            for b in range(NBUF):
                step(j0 + b, b)
            return carry

        lax.fori_loop(1, n_chunks // NBUF - 1, body, 0)

        step(n_chunks - 4, 0)
        step(n_chunks - 3, 1)
        gather(n_chunks - 2, 2).wait()
        write(n_chunks - 2, 2).start()
        gather(n_chunks - 1, 3).wait()
        write(n_chunks - 1, 3).start()
        write(n_chunks - 4, 0).wait()
        write(n_chunks - 3, 1).wait()
        write(n_chunks - 2, 2).wait()
        write(n_chunks - 1, 3).wait()

    out = sc_gather(idx3, table)
    return out.reshape(B, S, D)
